# baseline (speedup 1.0000x reference)
import functools

import jax
import jax.numpy as jnp
from jax import lax
from jax.experimental import pallas as pl
from jax.experimental.pallas import tpu as pltpu
from jax.experimental.pallas import tpu_sc as plsc

NC = 2
NS = 16
NW = NC * NS
LANES = 16
G = 11
IDXPAD = 128
NBUF = 2


def _full16(v):
    return jnp.full((LANES,), v, dtype=jnp.int32)


def _sc_gather_reduce(idx_all, w_all, feat_table, ng, d):
    k = (idx_all.shape[2] - G) // G
    gk = G * k
    rows_per_gather = IDXPAD
    bpad = NW * ng * G
    dsl = d // LANES

    mesh = plsc.VectorSubcoreMesh(core_axis_name="c", subcore_axis_name="s")

    @functools.partial(
        pl.kernel,
        mesh=mesh,
        compiler_params=pltpu.CompilerParams(needs_layout_passes=False),
        out_type=[
            jax.ShapeDtypeStruct((bpad * d,), jnp.float32),
            jax.ShapeDtypeStruct((bpad * d,), jnp.float32),
        ],
        scratch_types=(
            [pltpu.VMEM((ng, rows_per_gather), jnp.int32),
             pltpu.VMEM((ng * gk,), jnp.float32)]
            + [pltpu.VMEM((rows_per_gather, d), jnp.float32)
               for _ in range(NBUF)]
            + [pltpu.VMEM((G * d,), jnp.float32)
               for _ in range(2 * NBUF)]
            + [pltpu.SemaphoreType.DMA for _ in range(3 * NBUF)]
        ),
    )
    def sc_kernel(idx_hbm, w_hbm, table_hbm, neigh_hbm, self_hbm,
                  idx_sl, w_sl, *bufs):
        wid = lax.axis_index("s") * NC + lax.axis_index("c")
        rows_b = bufs[:NBUF]
        nout_b = bufs[NBUF:2 * NBUF]
        sst_b = bufs[2 * NBUF:3 * NBUF]
        gsem_b = bufs[3 * NBUF:4 * NBUF]
        nsem_b = bufs[4 * NBUF:5 * NBUF]
        ssem_b = bufs[5 * NBUF:6 * NBUF]

        pltpu.sync_copy(idx_hbm.at[wid], idx_sl)
        pltpu.sync_copy(w_hbm.at[wid], w_sl)

        def gather(g, p):
            return pltpu.make_async_copy(
                table_hbm.at[idx_sl.at[g]], rows_b[p], gsem_b[p])

        def out_copies(g, p):
            off = (wid * ng + g) * G * d
            nc = pltpu.make_async_copy(
                nout_b[p], neigh_hbm.at[pl.ds(off, G * d)], nsem_b[p])
            sc = pltpu.make_async_copy(
                sst_b[p], self_hbm.at[pl.ds(off, G * d)], ssem_b[p])
            return nc, sc

        for p0 in range(NBUF):
            gather(p0, p0).start()

        def step(g, p):
            rows, nout, sst = rows_b[p], nout_b[p], sst_b[p]
            gather(g, p).wait()

            @pl.when(g >= NBUF)
            def _():
                nc, sc = out_copies(g - NBUF, p)
                nc.wait()
                sc.wait()

            def body_b(b, _):
                base = b * k
                wbase = g * gk + base
                wv = [plsc.load_gather(w_sl, [_full16(wbase + j)])
                      for j in range(k)]
                wsum = wv[0]
                for j in range(1, k):
                    wsum = wsum + wv[j]
                inv = 1.0 / wsum
                for ds in range(dsl):
                    sl = pl.ds(ds * LANES, LANES)
                    osl = pl.ds(b * d + ds * LANES, LANES)
                    acc = wv[0] * rows[base, sl]
                    for j in range(1, k):
                        acc = acc + wv[j] * rows[base + j, sl]
                    nout[osl] = acc * inv
                    sst[osl] = rows[gk + b, sl]
                return 0

            lax.fori_loop(0, G, body_b, 0)

            nc, sc = out_copies(g, p)
            nc.start()
            sc.start()

            @pl.when(g + NBUF < ng)
            def _():
                gather(g + NBUF, p).start()

        def loop_body(i, _):
            for p in range(NBUF):
                step(NBUF * i + p, p)
            return 0

        lax.fori_loop(0, ng // NBUF, loop_body, 0)

        for p in range(NBUF):
            nc, sc = out_copies(ng - NBUF + p, p)
            nc.wait()
            sc.wait()

    return sc_kernel(idx_all, w_all, feat_table)


def _tc_dense(self_raw, neigh_feats, W_init, b_init, W_final, b_final, bm):
    bpad, d = self_raw.shape
    e = W_init.shape[1]

    def body(x_ref, n_ref, wi_ref, wf_ref, bi_ref, bf_ref, o_ref):
        sf = jnp.dot(x_ref[...], wi_ref[...],
                     preferred_element_type=jnp.float32) + bi_ref[...]
        out = (jnp.dot(sf, wf_ref[0:e, :], preferred_element_type=jnp.float32)
               + jnp.dot(n_ref[...], wf_ref[e:, :],
                         preferred_element_type=jnp.float32)
               + bf_ref[...])
        o_ref[...] = out * jax.nn.sigmoid(out)

    return pl.pallas_call(
        body,
        grid=(bpad // bm,),
        in_specs=[
            pl.BlockSpec((bm, d), lambda i: (i, 0)),
            pl.BlockSpec((bm, d), lambda i: (i, 0)),
            pl.BlockSpec(W_init.shape, lambda i: (0, 0)),
            pl.BlockSpec(W_final.shape, lambda i: (0, 0)),
            pl.BlockSpec((1, e), lambda i: (0, 0)),
            pl.BlockSpec((1, e), lambda i: (0, 0)),
        ],
        out_specs=pl.BlockSpec((bm, e), lambda i: (i, 0)),
        out_shape=jax.ShapeDtypeStruct((bpad, e), jnp.float32),
    )(self_raw, neigh_feats, W_init, W_final,
      b_init.reshape(1, e), b_final.reshape(1, e))


def kernel(nodes, neigh_idx, neigh_w, feat_table, W_init, b_init,
           W_final, b_final):
    b, k = neigh_idx.shape
    d = feat_table.shape[1]

    chunk = NW * G * NBUF
    bpad = ((b + chunk - 1) // chunk) * chunk
    ng = bpad // (NW * G)
    pad = bpad - b

    nodes_p = jnp.pad(nodes, (0, pad))
    nidx_p = jnp.pad(neigh_idx, ((0, pad), (0, 0)))
    w_p = jnp.pad(neigh_w, ((0, pad), (0, 0)), constant_values=1.0)

    nidx_g = nidx_p.reshape(NW, ng, G * k)
    nodes_g = nodes_p.reshape(NW, ng, G)
    idx_pad = jnp.zeros((NW, ng, IDXPAD - G * (k + 1)), jnp.int32)
    idx_all = jnp.concatenate([nidx_g, nodes_g, idx_pad], axis=2)
    w_all = w_p.reshape(NW, ng * G * k)

    neigh_flat, self_flat = _sc_gather_reduce(idx_all, w_all, feat_table,
                                              ng, d)
    neigh_feats = neigh_flat.reshape(bpad, d)
    self_raw = self_flat.reshape(bpad, d)
    out = _tc_dense(self_raw, neigh_feats, W_init, b_init, W_final, b_final,
                    bm=1024 if bpad % 1024 == 0 else 512)
    return out[:b]


# speedup vs baseline: 2.4098x; 2.4098x over previous
import functools

import jax
import jax.numpy as jnp
from jax import lax
from jax.experimental import pallas as pl
from jax.experimental.pallas import tpu as pltpu
from jax.experimental.pallas import tpu_sc as plsc

NC = 2
NS = 16
NW = NC * NS
LANES = 16
G = 11
IDXPAD = 128
NBUF = 2


def _full16(v):
    return jnp.full((LANES,), v, dtype=jnp.int32)


def _sc_gather_reduce(idx_all, w_all, feat_table, ng, d):
    k = (idx_all.shape[2] - G) // G
    gk = G * k
    rows_per_gather = IDXPAD
    bpad = NW * ng * G
    dsl = d // LANES

    mesh = plsc.VectorSubcoreMesh(core_axis_name="c", subcore_axis_name="s")

    @functools.partial(
        pl.kernel,
        mesh=mesh,
        compiler_params=pltpu.CompilerParams(needs_layout_passes=False),
        out_type=[
            jax.ShapeDtypeStruct((bpad * d,), jnp.float32),
            jax.ShapeDtypeStruct((bpad * d,), jnp.float32),
        ],
        scratch_types=(
            [pltpu.VMEM((ng, rows_per_gather), jnp.int32),
             pltpu.VMEM((ng * gk,), jnp.float32)]
            + [pltpu.VMEM((rows_per_gather, d), jnp.float32)
               for _ in range(NBUF)]
            + [pltpu.VMEM((G * d,), jnp.float32)
               for _ in range(2 * NBUF)]
            + [pltpu.SemaphoreType.DMA for _ in range(3 * NBUF)]
        ),
    )
    def sc_kernel(idx_hbm, w_hbm, table_hbm, neigh_hbm, self_hbm,
                  idx_sl, w_sl, *bufs):
        wid = lax.axis_index("s") * NC + lax.axis_index("c")
        rows_b = bufs[:NBUF]
        nout_b = bufs[NBUF:2 * NBUF]
        sst_b = bufs[2 * NBUF:3 * NBUF]
        gsem_b = bufs[3 * NBUF:4 * NBUF]
        nsem_b = bufs[4 * NBUF:5 * NBUF]
        ssem_b = bufs[5 * NBUF:6 * NBUF]

        pltpu.sync_copy(idx_hbm.at[wid], idx_sl)
        pltpu.sync_copy(w_hbm.at[wid], w_sl)

        def gather(g, p):
            return pltpu.make_async_copy(
                table_hbm.at[idx_sl.at[g]], rows_b[p], gsem_b[p])

        def out_copies(g, p):
            off = (wid * ng + g) * G * d
            nc = pltpu.make_async_copy(
                nout_b[p], neigh_hbm.at[pl.ds(off, G * d)], nsem_b[p])
            sc = pltpu.make_async_copy(
                sst_b[p], self_hbm.at[pl.ds(off, G * d)], ssem_b[p])
            return nc, sc

        for p0 in range(NBUF):
            gather(p0, p0).start()

        def step(g, p):
            rows, nout, sst = rows_b[p], nout_b[p], sst_b[p]
            gather(g, p).wait()

            @pl.when(g >= NBUF)
            def _():
                nc, sc = out_copies(g - NBUF, p)
                nc.wait()
                sc.wait()

            def body_b(b, _):
                base = b * k
                wbase = g * gk + base
                wv = [plsc.load_gather(w_sl, [_full16(wbase + j)])
                      for j in range(k)]
                wsum = wv[0]
                for j in range(1, k):
                    wsum = wsum + wv[j]
                inv = 1.0 / wsum
                for ds in range(dsl):
                    sl = pl.ds(ds * LANES, LANES)
                    osl = pl.ds(b * d + ds * LANES, LANES)
                    acc = wv[0] * rows[base, sl]
                    for j in range(1, k):
                        acc = acc + wv[j] * rows[base + j, sl]
                    nout[osl] = acc * inv
                    sst[osl] = rows[gk + b, sl]
                return 0

            lax.fori_loop(0, G, body_b, 0)

            nc, sc = out_copies(g, p)
            nc.start()
            sc.start()

            @pl.when(g + NBUF < ng)
            def _():
                gather(g + NBUF, p).start()

        def loop_body(i, _):
            for p in range(NBUF):
                step(NBUF * i + p, p)
            return 0

        lax.fori_loop(0, ng // NBUF, loop_body, 0)

        for p in range(NBUF):
            nc, sc = out_copies(ng - NBUF + p, p)
            nc.wait()
            sc.wait()

    return sc_kernel(idx_all, w_all, feat_table)


def _tc_dense(self_raw, neigh_feats, W_init, b_init, W_final, b_final, bm):
    bpad, d = self_raw.shape
    e = W_init.shape[1]

    def body(x_ref, n_ref, wi_ref, wf_ref, bi_ref, bf_ref, o_ref):
        sf = jnp.dot(x_ref[...], wi_ref[...],
                     preferred_element_type=jnp.float32) + bi_ref[...]
        out = (jnp.dot(sf, wf_ref[0:e, :], preferred_element_type=jnp.float32)
               + jnp.dot(n_ref[...], wf_ref[e:, :],
                         preferred_element_type=jnp.float32)
               + bf_ref[...])
        o_ref[...] = out * jax.nn.sigmoid(out)

    return pl.pallas_call(
        body,
        grid=(bpad // bm,),
        in_specs=[
            pl.BlockSpec((bm, d), lambda i: (i, 0)),
            pl.BlockSpec((bm, d), lambda i: (i, 0)),
            pl.BlockSpec(W_init.shape, lambda i: (0, 0)),
            pl.BlockSpec(W_final.shape, lambda i: (0, 0)),
            pl.BlockSpec((1, e), lambda i: (0, 0)),
            pl.BlockSpec((1, e), lambda i: (0, 0)),
        ],
        out_specs=pl.BlockSpec((bm, e), lambda i: (i, 0)),
        out_shape=jax.ShapeDtypeStruct((bpad, e), jnp.float32),
    )(self_raw, neigh_feats, W_init, W_final,
      b_init.reshape(1, e), b_final.reshape(1, e))


def kernel(nodes, neigh_idx, neigh_w, feat_table, W_init, b_init,
           W_final, b_final):
    b, k = neigh_idx.shape
    d = feat_table.shape[1]

    chunk = NW * G * NBUF
    bpad = ((b + chunk - 1) // chunk) * chunk
    ng = bpad // (NW * G)
    pad = bpad - b

    nodes_p = jnp.pad(nodes, (0, pad))
    nidx_p = jnp.pad(neigh_idx, ((0, pad), (0, 0)))
    w_p = jnp.pad(neigh_w, ((0, pad), (0, 0)), constant_values=1.0)

    nidx_g = nidx_p.reshape(NW, ng, G * k)
    nodes_g = nodes_p.reshape(NW, ng, G)
    npad_idx = IDXPAD - G * (k + 1)
    n_nodes = feat_table.shape[0]
    idx_pad = (
        jax.lax.broadcasted_iota(jnp.int32, (NW, ng, npad_idx), 0) * 8191
        + jax.lax.broadcasted_iota(jnp.int32, (NW, ng, npad_idx), 1) * 131
        + jax.lax.broadcasted_iota(jnp.int32, (NW, ng, npad_idx), 2)
    ) % n_nodes
    idx_all = jnp.concatenate([nidx_g, nodes_g, idx_pad], axis=2)
    w_all = w_p.reshape(NW, ng * G * k)

    neigh_flat, self_flat = _sc_gather_reduce(idx_all, w_all, feat_table,
                                              ng, d)
    neigh_feats = neigh_flat.reshape(bpad, d)
    self_raw = self_flat.reshape(bpad, d)
    out = _tc_dense(self_raw, neigh_feats, W_init, b_init, W_final, b_final,
                    bm=1024 if bpad % 1024 == 0 else 512)
    return out[:b]


# speedup vs baseline: 4.4314x; 1.8389x over previous
import functools

import jax
import jax.numpy as jnp
from jax import lax
from jax.experimental import pallas as pl
from jax.experimental.pallas import tpu as pltpu
from jax.experimental.pallas import tpu_sc as plsc

NC = 2
NS = 16
NW = NC * NS
LANES = 16
G = 8
NBUF = 2


def _full16(v):
    return jnp.full((LANES,), v, dtype=jnp.int32)


def _sc_gather_reduce(nidx, nodes, w, feat_table, ng, d, k):
    gk = G * k
    bpad = NW * ng * G
    dsl = d // LANES

    mesh = plsc.VectorSubcoreMesh(core_axis_name="c", subcore_axis_name="s")

    @functools.partial(
        pl.kernel,
        mesh=mesh,
        compiler_params=pltpu.CompilerParams(needs_layout_passes=False),
        out_type=[
            jax.ShapeDtypeStruct((bpad, d), jnp.float32),
            jax.ShapeDtypeStruct((bpad, d), jnp.float32),
        ],
        scratch_types=(
            [pltpu.VMEM((ng * gk,), jnp.int32),
             pltpu.VMEM((ng * G,), jnp.int32),
             pltpu.VMEM((ng * gk,), jnp.float32)]
            + [pltpu.VMEM((gk, d), jnp.float32) for _ in range(NBUF)]
            + [pltpu.VMEM((G, d), jnp.float32) for _ in range(NBUF)]
            + [pltpu.VMEM((G, d), jnp.float32) for _ in range(NBUF)]
            + [pltpu.SemaphoreType.DMA for _ in range(4 * NBUF)]
        ),
    )
    def sc_kernel(nidx_hbm, nodes_hbm, w_hbm, table_hbm, neigh_hbm, self_hbm,
                  nidx_sl, nodes_sl, w_sl, *bufs):
        wid = lax.axis_index("s") * NC + lax.axis_index("c")
        rows_b = bufs[:NBUF]
        sst_b = bufs[NBUF:2 * NBUF]
        nout_b = bufs[2 * NBUF:3 * NBUF]
        gsem_b = bufs[3 * NBUF:4 * NBUF]
        sgsem_b = bufs[4 * NBUF:5 * NBUF]
        nsem_b = bufs[5 * NBUF:6 * NBUF]
        ssem_b = bufs[6 * NBUF:7 * NBUF]

        pltpu.sync_copy(nidx_hbm.at[wid], nidx_sl)
        pltpu.sync_copy(nodes_hbm.at[wid], nodes_sl)
        pltpu.sync_copy(w_hbm.at[wid], w_sl)

        def big_gather(g, p):
            return pltpu.make_async_copy(
                table_hbm.at[nidx_sl.at[pl.ds(g * gk, gk)]],
                rows_b[p], gsem_b[p])

        def self_gather(g, p):
            return pltpu.make_async_copy(
                table_hbm.at[nodes_sl.at[pl.ds(g * G, G)]],
                sst_b[p], sgsem_b[p])

        def out_copies(g, p):
            row0 = (wid * ng + g) * G
            nc = pltpu.make_async_copy(
                nout_b[p], neigh_hbm.at[pl.ds(row0, G), :], nsem_b[p])
            sc = pltpu.make_async_copy(
                sst_b[p], self_hbm.at[pl.ds(row0, G), :], ssem_b[p])
            return nc, sc

        for p0 in range(NBUF):
            big_gather(p0, p0).start()
            self_gather(p0, p0).start()

        def step(g, p):
            rows, nout = rows_b[p], nout_b[p]

            @pl.when(g >= NBUF)
            def _():
                nc, sc = out_copies(g - NBUF, p)
                nc.wait()
                sc.wait()
                self_gather(g, p).start()

            big_gather(g, p).wait()

            def body_b(b, _):
                base = b * k
                wbase = g * gk + base
                wv = [plsc.load_gather(w_sl, [_full16(wbase + j)])
                      for j in range(k)]
                wsum = wv[0]
                for j in range(1, k):
                    wsum = wsum + wv[j]
                inv = 1.0 / wsum
                for ds in range(dsl):
                    sl = pl.ds(ds * LANES, LANES)
                    acc = wv[0] * rows[base, sl]
                    for j in range(1, k):
                        acc = acc + wv[j] * rows[base + j, sl]
                    nout[b, sl] = acc * inv
                return 0

            lax.fori_loop(0, G, body_b, 0)

            self_gather(g, p).wait()
            nc, sc = out_copies(g, p)
            nc.start()
            sc.start()

            @pl.when(g + NBUF < ng)
            def _():
                big_gather(g + NBUF, p).start()

        def loop_body(i, _):
            for p in range(NBUF):
                step(NBUF * i + p, p)
            return 0

        lax.fori_loop(0, ng // NBUF, loop_body, 0)

        for p in range(NBUF):
            nc, sc = out_copies(ng - NBUF + p, p)
            nc.wait()
            sc.wait()

    return sc_kernel(nidx, nodes, w, feat_table)


def _tc_dense(self_raw, neigh_feats, W_init, b_init, W_final, b_final,
              bm, b_rows):
    bpad, d = self_raw.shape
    e = W_init.shape[1]

    def body(x_ref, n_ref, wi_ref, wf_ref, bi_ref, bf_ref, o_ref):
        wc = jnp.dot(wi_ref[...], wf_ref[0:e, :],
                     preferred_element_type=jnp.float32)
        bias = jnp.dot(bi_ref[...], wf_ref[0:e, :],
                       preferred_element_type=jnp.float32) + bf_ref[...]
        out = (jnp.dot(x_ref[...], wc, preferred_element_type=jnp.float32)
               + jnp.dot(n_ref[...], wf_ref[e:, :],
                         preferred_element_type=jnp.float32)
               + bias)
        o_ref[...] = out * jax.nn.sigmoid(out)

    return pl.pallas_call(
        body,
        grid=(bpad // bm,),
        in_specs=[
            pl.BlockSpec((bm, d), lambda i: (i, 0)),
            pl.BlockSpec((bm, d), lambda i: (i, 0)),
            pl.BlockSpec(W_init.shape, lambda i: (0, 0)),
            pl.BlockSpec(W_final.shape, lambda i: (0, 0)),
            pl.BlockSpec((1, e), lambda i: (0, 0)),
            pl.BlockSpec((1, e), lambda i: (0, 0)),
        ],
        out_specs=pl.BlockSpec((bm, e), lambda i: (i, 0)),
        out_shape=jax.ShapeDtypeStruct((b_rows, e), jnp.float32),
    )(self_raw, neigh_feats, W_init, W_final,
      b_init.reshape(1, e), b_final.reshape(1, e))


def kernel(nodes, neigh_idx, neigh_w, feat_table, W_init, b_init,
           W_final, b_final):
    b, k = neigh_idx.shape
    d = feat_table.shape[1]

    chunk = NW * G * NBUF
    bpad = ((b + chunk - 1) // chunk) * chunk
    ng = bpad // (NW * G)
    pad = bpad - b

    nidx_f = jnp.pad(neigh_idx.reshape(-1), (0, pad * k)).reshape(NW, ng * G * k)
    w_f = jnp.pad(neigh_w.reshape(-1), (0, pad * k),
                  constant_values=1.0).reshape(NW, ng * G * k)
    nodes_f = jnp.pad(nodes, (0, pad)).reshape(NW, ng * G)

    neigh_feats, self_raw = _sc_gather_reduce(nidx_f, nodes_f, w_f,
                                              feat_table, ng, d, k)
    return _tc_dense(self_raw, neigh_feats, W_init, b_init, W_final, b_final,
                     bm=1024 if bpad % 1024 == 0 else 512, b_rows=b)
